# baseline (speedup 1.0000x reference)
import functools

import jax
import jax.numpy as jnp
from jax import lax
from jax.experimental import pallas as pl
from jax.experimental.pallas import tpu as pltpu
from jax.experimental.pallas import tpu_sc as plsc

NN = 32
NQ = 1024
DIM = 64
NK = 100000
KB = 2048
NBLK = 50
K_PAD = KB * NBLK
NC, NS, L = 2, 16, 16
NW = NC * NS
ROWS_PER_W = NQ // NW
CH = 25600
NCH = K_PAD // CH
R = 16
GROUP = R * L
NGROUPS = CH // GROUP
INF = float("inf")
BIGI = 2**30


def _dist_body(q_ref, k_ref, o_ref):
    j = pl.program_id(0)
    q = q_ref[...]
    k = k_ref[...]
    q_sq = jnp.sum(q * q, axis=1)
    k_sq = jnp.sum(k * k, axis=1)
    dot = lax.dot_general(q, k, (((1,), (1,)), ((), ())),
                          preferred_element_type=jnp.float32)
    d = (q_sq[:, None] - 2.0 * dot) + k_sq[None, :]
    col = j * KB + jax.lax.broadcasted_iota(jnp.int32, d.shape, 1)
    o_ref[...] = jnp.where(col < NK, d, INF)


def _distances(queries, keys):
    keys_pad = jnp.concatenate(
        [keys, jnp.zeros((K_PAD - NK, DIM), keys.dtype)], axis=0)
    return pl.pallas_call(
        _dist_body,
        grid=(NBLK,),
        in_specs=[
            pl.BlockSpec((NQ, DIM), lambda j: (0, 0)),
            pl.BlockSpec((KB, DIM), lambda j: (j, 0)),
        ],
        out_specs=pl.BlockSpec((NQ, KB), lambda j: (0, j)),
        out_shape=jax.ShapeDtypeStruct((NQ, K_PAD), jnp.float32),
    )(queries, keys_pad)


def _splat(x):
    return jnp.broadcast_to(x, (L,))


def _merge16(state, v, base):
    A, Ai, B, Bi, tau = state
    iota = lax.broadcasted_iota(jnp.int32, (L,), 0)
    msk = v < tau
    vc = jnp.where(msk, v, INF)
    ic = jnp.where(msk, _splat(base) + iota, BIGI)
    vc, ic = plsc.sort_key_val(vc, ic)
    rv = lax.rev(vc, (0,))
    ri = lax.rev(ic, (0,))
    m = A <= rv
    lo = jnp.where(m, A, rv)
    loi = jnp.where(m, Ai, ri)
    hi = jnp.where(m, rv, A)
    hii = jnp.where(m, ri, Ai)
    A2, Ai2 = plsc.sort_key_val(lo, loi)
    hs, hsi = plsc.sort_key_val(hi, hii)
    rh = lax.rev(hs, (0,))
    rhi = lax.rev(hsi, (0,))
    m2 = B <= rh
    lo2 = jnp.where(m2, B, rh)
    lo2i = jnp.where(m2, Bi, rhi)
    B2, Bi2 = plsc.sort_key_val(lo2, lo2i)
    tau2 = B2.at[jnp.full((L,), L - 1, jnp.int32)].get(mode="promise_in_bounds")
    return A2, Ai2, B2, Bi2, tau2


def _sc_topk_body(dist_hbm, vals_hbm, idx_hbm, buf_a, buf_b, ov, oi,
                  sem_a, sem_b):
    wid = lax.axis_index("s") * NC + lax.axis_index("c")
    row0 = wid * ROWS_PER_W

    def do_row(r, _):
        row = row0 + r

        pltpu.make_async_copy(
            dist_hbm.at[row, pl.ds(0, CH)], buf_a, sem_a).start()

        def group_step(cbase, buf):
            def step(g, state):
                off = pl.multiple_of(g * GROUP, GROUP)
                vs = [buf[pl.ds(off + k * L, L)] for k in range(R)]
                gm = vs[0]
                for k in range(1, R):
                    gm = jnp.minimum(gm, vs[k])

                def slow(st):
                    for k in range(R):
                        v = vs[k]
                        base = cbase + off + k * L
                        hit = jnp.any(v < st[4])
                        st = lax.cond(
                            hit, functools.partial(_merge16, v=v, base=base),
                            lambda s: s, st)
                    return st

                return lax.cond(jnp.any(gm < state[4]), slow,
                                lambda s: s, state)
            return step

        state = (
            jnp.full((L,), INF, jnp.float32), jnp.full((L,), BIGI, jnp.int32),
            jnp.full((L,), INF, jnp.float32), jnp.full((L,), BIGI, jnp.int32),
            jnp.full((L,), INF, jnp.float32),
        )
        for c in range(NCH):
            buf, sem = (buf_a, sem_a) if c % 2 == 0 else (buf_b, sem_b)
            pltpu.make_async_copy(
                dist_hbm.at[row, pl.ds(c * CH, CH)], buf, sem).wait()
            if c + 1 < NCH:
                nbuf, nsem = (buf_b, sem_b) if c % 2 == 0 else (buf_a, sem_a)
                pltpu.make_async_copy(
                    dist_hbm.at[row, pl.ds((c + 1) * CH, CH)],
                    nbuf, nsem).start()
            state = lax.fori_loop(0, NGROUPS, group_step(c * CH, buf), state)

        A, Ai, B, Bi, _ = state
        ov[pl.ds(0, L)] = A
        ov[pl.ds(L, L)] = B
        oi[pl.ds(0, L)] = Ai
        oi[pl.ds(L, L)] = Bi
        obase = pl.multiple_of(row * NN, NN)
        pltpu.sync_copy(ov, vals_hbm.at[pl.ds(obase, NN)])
        pltpu.sync_copy(oi, idx_hbm.at[pl.ds(obase, NN)])
        return 0

    lax.fori_loop(0, ROWS_PER_W, do_row, 0)


def _sc_topk(dist):
    mesh = plsc.VectorSubcoreMesh(core_axis_name="c", subcore_axis_name="s")
    kern = pl.kernel(
        _sc_topk_body,
        out_type=[
            jax.ShapeDtypeStruct((NQ * NN,), jnp.float32),
            jax.ShapeDtypeStruct((NQ * NN,), jnp.int32),
        ],
        mesh=mesh,
        scratch_types=[
            pltpu.VMEM((CH,), jnp.float32),
            pltpu.VMEM((CH,), jnp.float32),
            pltpu.VMEM((NN,), jnp.float32),
            pltpu.VMEM((NN,), jnp.int32),
            pltpu.SemaphoreType.DMA,
            pltpu.SemaphoreType.DMA,
        ],
        compiler_params=pltpu.CompilerParams(needs_layout_passes=False),
    )
    return kern(dist)


@jax.jit
def kernel(queries, keys):
    dist = _distances(queries, keys)
    vals, idx = _sc_topk(dist)
    return vals.reshape(NQ, NN), idx.reshape(NQ, NN)


# speedup vs baseline: 1.2540x; 1.2540x over previous
import functools

import jax
import jax.numpy as jnp
from jax import lax
from jax.experimental import pallas as pl
from jax.experimental.pallas import tpu as pltpu
from jax.experimental.pallas import tpu_sc as plsc

NN = 32
NQ = 1024
DIM = 64
NK = 100000
KB = 2048
NBLK = 50
K_PAD = KB * NBLK
NC, NS, L = 2, 16, 16
NW = NC * NS
ROWS_PER_W = NQ // NW
CH = 25600
NCH = K_PAD // CH
R = 8
GROUP = R * L
NGROUPS = CH // GROUP
INF = float("inf")
BIGI = 2**30


def _dist_body(q_ref, k_ref, o_ref):
    j = pl.program_id(0)
    q = q_ref[...]
    k = k_ref[...]
    q_sq = jnp.sum(q * q, axis=1)
    k_sq = jnp.sum(k * k, axis=1)
    dot = lax.dot_general(q, k, (((1,), (1,)), ((), ())),
                          preferred_element_type=jnp.float32)
    d = (q_sq[:, None] - 2.0 * dot) + k_sq[None, :]
    col = j * KB + jax.lax.broadcasted_iota(jnp.int32, d.shape, 1)
    o_ref[...] = jnp.where(col < NK, d, INF)


def _distances(queries, keys):
    keys_pad = jnp.concatenate(
        [keys, jnp.zeros((K_PAD - NK, DIM), keys.dtype)], axis=0)
    return pl.pallas_call(
        _dist_body,
        grid=(NBLK,),
        in_specs=[
            pl.BlockSpec((NQ, DIM), lambda j: (0, 0)),
            pl.BlockSpec((KB, DIM), lambda j: (j, 0)),
        ],
        out_specs=pl.BlockSpec((NQ, KB), lambda j: (0, j)),
        out_shape=jax.ShapeDtypeStruct((NQ, K_PAD), jnp.float32),
    )(queries, keys_pad)


def _splat(x):
    return jnp.broadcast_to(x, (L,))


def _merge16(state, v, base):
    A, Ai, B, Bi, tau = state
    iota = lax.broadcasted_iota(jnp.int32, (L,), 0)
    msk = v < tau
    vc = jnp.where(msk, v, INF)
    ic = jnp.where(msk, _splat(base) + iota, BIGI)
    vc, ic = plsc.sort_key_val(vc, ic)
    rv = lax.rev(vc, (0,))
    ri = lax.rev(ic, (0,))
    m = A <= rv
    lo = jnp.where(m, A, rv)
    loi = jnp.where(m, Ai, ri)
    hi = jnp.where(m, rv, A)
    hii = jnp.where(m, ri, Ai)
    A2, Ai2 = plsc.sort_key_val(lo, loi)
    hs, hsi = plsc.sort_key_val(hi, hii)
    rh = lax.rev(hs, (0,))
    rhi = lax.rev(hsi, (0,))
    m2 = B <= rh
    lo2 = jnp.where(m2, B, rh)
    lo2i = jnp.where(m2, Bi, rhi)
    B2, Bi2 = plsc.sort_key_val(lo2, lo2i)
    tau2 = B2.at[jnp.full((L,), L - 1, jnp.int32)].get(mode="promise_in_bounds")
    return A2, Ai2, B2, Bi2, tau2


def _sc_topk_body(dist_hbm, vals_hbm, idx_hbm, buf_a, buf_b, ov, oi,
                  sem_a, sem_b):
    wid = lax.axis_index("s") * NC + lax.axis_index("c")
    row0 = wid * ROWS_PER_W

    def do_row(r, _):
        row = row0 + r

        pltpu.make_async_copy(
            dist_hbm.at[row, pl.ds(0, CH)], buf_a, sem_a).start()

        def group_step(cbase, buf):
            def step(g, state):
                off = pl.multiple_of(g * GROUP, GROUP)
                vs = [buf[pl.ds(off + k * L, L)] for k in range(R)]
                gm = vs[0]
                for k in range(1, R):
                    gm = jnp.minimum(gm, vs[k])

                def slow(st):
                    for k in range(R):
                        v = vs[k]
                        base = cbase + off + k * L
                        hit = jnp.any(v < st[4])
                        st = lax.cond(
                            hit, functools.partial(_merge16, v=v, base=base),
                            lambda s: s, st)
                    return st

                return lax.cond(jnp.any(gm < state[4]), slow,
                                lambda s: s, state)
            return step

        state = (
            jnp.full((L,), INF, jnp.float32), jnp.full((L,), BIGI, jnp.int32),
            jnp.full((L,), INF, jnp.float32), jnp.full((L,), BIGI, jnp.int32),
            jnp.full((L,), INF, jnp.float32),
        )
        for c in range(NCH):
            buf, sem = (buf_a, sem_a) if c % 2 == 0 else (buf_b, sem_b)
            pltpu.make_async_copy(
                dist_hbm.at[row, pl.ds(c * CH, CH)], buf, sem).wait()
            if c + 1 < NCH:
                nbuf, nsem = (buf_b, sem_b) if c % 2 == 0 else (buf_a, sem_a)
                pltpu.make_async_copy(
                    dist_hbm.at[row, pl.ds((c + 1) * CH, CH)],
                    nbuf, nsem).start()
            state = lax.fori_loop(0, NGROUPS, group_step(c * CH, buf), state)

        A, Ai, B, Bi, _ = state
        ov[pl.ds(0, L)] = A
        ov[pl.ds(L, L)] = B
        oi[pl.ds(0, L)] = Ai
        oi[pl.ds(L, L)] = Bi
        obase = pl.multiple_of(row * NN, NN)
        pltpu.sync_copy(ov, vals_hbm.at[pl.ds(obase, NN)])
        pltpu.sync_copy(oi, idx_hbm.at[pl.ds(obase, NN)])
        return 0

    lax.fori_loop(0, ROWS_PER_W, do_row, 0)


def _sc_topk(dist):
    mesh = plsc.VectorSubcoreMesh(core_axis_name="c", subcore_axis_name="s")
    kern = pl.kernel(
        _sc_topk_body,
        out_type=[
            jax.ShapeDtypeStruct((NQ * NN,), jnp.float32),
            jax.ShapeDtypeStruct((NQ * NN,), jnp.int32),
        ],
        mesh=mesh,
        scratch_types=[
            pltpu.VMEM((CH,), jnp.float32),
            pltpu.VMEM((CH,), jnp.float32),
            pltpu.VMEM((NN,), jnp.float32),
            pltpu.VMEM((NN,), jnp.int32),
            pltpu.SemaphoreType.DMA,
            pltpu.SemaphoreType.DMA,
        ],
        compiler_params=pltpu.CompilerParams(needs_layout_passes=False),
    )
    return kern(dist)


@jax.jit
def kernel(queries, keys):
    dist = _distances(queries, keys)
    vals, idx = _sc_topk(dist)
    return vals.reshape(NQ, NN), idx.reshape(NQ, NN)


# speedup vs baseline: 1.2595x; 1.0044x over previous
import functools

import jax
import jax.numpy as jnp
from jax import lax
from jax.experimental import pallas as pl
from jax.experimental.pallas import tpu as pltpu
from jax.experimental.pallas import tpu_sc as plsc

NN = 32
NQ = 1024
DIM = 64
NK = 100000
KB = 2048
NBLK = 50
K_PAD = KB * NBLK
NC, NS, L = 2, 16, 16
NW = NC * NS
ROWS_PER_W = NQ // NW
CH = 20480
NCH = K_PAD // CH
BPC = CH // KB
R = 8
SEG = KB // R
INF = float("inf")
BIGI = 2**30


KGM = K_PAD // R


def _dist_body(q_ref, k_ref, o_ref, g_ref):
    j = pl.program_id(0)
    q = q_ref[...]
    k = k_ref[...]
    q_sq = jnp.sum(q * q, axis=1)
    k_sq = jnp.sum(k * k, axis=1)
    dot = lax.dot_general(q, k, (((1,), (1,)), ((), ())),
                          preferred_element_type=jnp.float32)
    d = (q_sq[:, None] - 2.0 * dot) + k_sq[None, :]
    col = j * KB + jax.lax.broadcasted_iota(jnp.int32, d.shape, 1)
    d = jnp.where(col < NK, d, INF)
    o_ref[...] = d
    g = d[:, 0:SEG]
    for t in range(1, R):
        g = jnp.minimum(g, d[:, t * SEG:(t + 1) * SEG])
    g_ref[...] = g


def _distances(queries, keys):
    keys_pad = jnp.concatenate(
        [keys, jnp.zeros((K_PAD - NK, DIM), keys.dtype)], axis=0)
    return pl.pallas_call(
        _dist_body,
        grid=(NBLK,),
        in_specs=[
            pl.BlockSpec((NQ, DIM), lambda j: (0, 0)),
            pl.BlockSpec((KB, DIM), lambda j: (j, 0)),
        ],
        out_specs=[
            pl.BlockSpec((NQ, KB), lambda j: (0, j)),
            pl.BlockSpec((NQ, SEG), lambda j: (0, j)),
        ],
        out_shape=[
            jax.ShapeDtypeStruct((NQ, K_PAD), jnp.float32),
            jax.ShapeDtypeStruct((NQ, KGM), jnp.float32),
        ],
    )(queries, keys_pad)


def _splat(x):
    return jnp.broadcast_to(x, (L,))


def _merge16(state, v, base):
    A, Ai, B, Bi, tau = state
    iota = lax.broadcasted_iota(jnp.int32, (L,), 0)
    msk = v < tau
    vc = jnp.where(msk, v, INF)
    ic = jnp.where(msk, _splat(base) + iota, BIGI)
    vc, ic = plsc.sort_key_val(vc, ic)
    rv = lax.rev(vc, (0,))
    ri = lax.rev(ic, (0,))
    m = A <= rv
    lo = jnp.where(m, A, rv)
    loi = jnp.where(m, Ai, ri)
    hi = jnp.where(m, rv, A)
    hii = jnp.where(m, ri, Ai)
    A2, Ai2 = plsc.sort_key_val(lo, loi)
    hs, hsi = plsc.sort_key_val(hi, hii)
    rh = lax.rev(hs, (0,))
    rhi = lax.rev(hsi, (0,))
    m2 = B <= rh
    lo2 = jnp.where(m2, B, rh)
    lo2i = jnp.where(m2, Bi, rhi)
    B2, Bi2 = plsc.sort_key_val(lo2, lo2i)
    tau2 = B2.at[jnp.full((L,), L - 1, jnp.int32)].get(mode="promise_in_bounds")
    return A2, Ai2, B2, Bi2, tau2


def _sc_topk_body(dist_hbm, gmin_hbm, vals_hbm, idx_hbm, buf_a, buf_b, gbuf,
                  ov, oi, sem_a, sem_b, sem_g):
    wid = lax.axis_index("s") * NC + lax.axis_index("c")
    row0 = wid * ROWS_PER_W

    def do_row(r, _):
        row = row0 + r

        pltpu.make_async_copy(
            gmin_hbm.at[row, pl.ds(0, KGM)], gbuf, sem_g).start()
        pltpu.make_async_copy(
            dist_hbm.at[row, pl.ds(0, CH)], buf_a, sem_a).start()
        pltpu.make_async_copy(
            gmin_hbm.at[row, pl.ds(0, KGM)], gbuf, sem_g).wait()

        def blk_step(cbase, buf):
            def bstep(blk, state):
                boff = pl.multiple_of(blk * KB, KB)

                def gstep(gg, st):
                    goff = pl.multiple_of(gg * L, L)
                    gm = gbuf[pl.ds(cbase // R + blk * SEG + goff, L)]

                    def slow(s2):
                        for t in range(R):
                            v = buf[pl.ds(boff + t * SEG + goff, L)]
                            base = cbase + boff + t * SEG + goff
                            hit = jnp.any(v < s2[4])
                            s2 = lax.cond(
                                hit,
                                functools.partial(_merge16, v=v, base=base),
                                lambda s: s, s2)
                        return s2

                    return lax.cond(jnp.any(gm < st[4]), slow,
                                    lambda s: s, st)

                return lax.fori_loop(0, SEG // L, gstep, state)
            return bstep

        state = (
            jnp.full((L,), INF, jnp.float32), jnp.full((L,), BIGI, jnp.int32),
            jnp.full((L,), INF, jnp.float32), jnp.full((L,), BIGI, jnp.int32),
            jnp.full((L,), INF, jnp.float32),
        )
        for c in range(NCH):
            buf, sem = (buf_a, sem_a) if c % 2 == 0 else (buf_b, sem_b)
            pltpu.make_async_copy(
                dist_hbm.at[row, pl.ds(c * CH, CH)], buf, sem).wait()
            if c + 1 < NCH:
                nbuf, nsem = (buf_b, sem_b) if c % 2 == 0 else (buf_a, sem_a)
                pltpu.make_async_copy(
                    dist_hbm.at[row, pl.ds((c + 1) * CH, CH)],
                    nbuf, nsem).start()
            state = lax.fori_loop(0, BPC, blk_step(c * CH, buf), state)

        A, Ai, B, Bi, _ = state
        ov[pl.ds(0, L)] = A
        ov[pl.ds(L, L)] = B
        oi[pl.ds(0, L)] = Ai
        oi[pl.ds(L, L)] = Bi
        obase = pl.multiple_of(row * NN, NN)
        pltpu.sync_copy(ov, vals_hbm.at[pl.ds(obase, NN)])
        pltpu.sync_copy(oi, idx_hbm.at[pl.ds(obase, NN)])
        return 0

    lax.fori_loop(0, ROWS_PER_W, do_row, 0)


def _sc_topk(dist, gmin):
    mesh = plsc.VectorSubcoreMesh(core_axis_name="c", subcore_axis_name="s")
    kern = pl.kernel(
        _sc_topk_body,
        out_type=[
            jax.ShapeDtypeStruct((NQ * NN,), jnp.float32),
            jax.ShapeDtypeStruct((NQ * NN,), jnp.int32),
        ],
        mesh=mesh,
        scratch_types=[
            pltpu.VMEM((CH,), jnp.float32),
            pltpu.VMEM((CH,), jnp.float32),
            pltpu.VMEM((KGM,), jnp.float32),
            pltpu.VMEM((NN,), jnp.float32),
            pltpu.VMEM((NN,), jnp.int32),
            pltpu.SemaphoreType.DMA,
            pltpu.SemaphoreType.DMA,
            pltpu.SemaphoreType.DMA,
        ],
        compiler_params=pltpu.CompilerParams(needs_layout_passes=False),
    )
    return kern(dist, gmin)


@jax.jit
def kernel(queries, keys):
    dist, gmin = _distances(queries, keys)
    vals, idx = _sc_topk(dist, gmin)
    return vals.reshape(NQ, NN), idx.reshape(NQ, NN)
